# baseline (speedup 1.0000x reference)
import jax
import jax.numpy as jnp
from jax import lax
from jax.experimental import pallas as pl
from jax.experimental.pallas import tpu as pltpu
from jax.experimental.pallas import tpu_sc as plsc

B = 16384
FACTOR = 32
F2 = 2 * FACTOR
ENV_NUM = 4
NC = 2
NS = 16
L = 16
NW = NC * NS
CHUNK = B // NW
NBLK = CHUNK // L
GCH = 128
NG = CHUNK // GCH

_LC0 = -0.76336156
_LC1 = 0.9123227
_LC2 = -0.09557938


def _sc_body(users_hbm, items_hbm, envs_hbm,
             utab_hbm, itab_hbm,
             env_hbm, w_hbm, b_hbm,
             inv_hbm, envsc_hbm, lsm_hbm,
             uidx_v, iidx_v, envs_v,
             urow_v, irow_v,
             envt_v, w_v, b_v,
             inv_v, envsc_v, lsm_v, sem):
    wid = lax.axis_index("s") * NC + lax.axis_index("c")
    base = wid * CHUNK
    row4 = wid * NG

    pltpu.sync_copy(users_hbm.at[pl.ds(row4, NG)], uidx_v)
    pltpu.sync_copy(items_hbm.at[pl.ds(row4, NG)], iidx_v)
    pltpu.sync_copy(envs_hbm.at[pl.ds(base, CHUNK)], envs_v)
    pltpu.sync_copy(env_hbm, envt_v)
    pltpu.sync_copy(w_hbm, w_v)
    pltpu.sync_copy(b_hbm, b_v)

    copies = []
    for j in range(NG):
        dst = pl.ds(j * GCH, GCH)
        copies.append(pltpu.async_copy(utab_hbm.at[uidx_v.at[j]], urow_v.at[dst], sem))
        copies.append(pltpu.async_copy(itab_hbm.at[iidx_v.at[j]], irow_v.at[dst], sem))
    for c in copies:
        c.wait()

    wrows = [w_v[pl.ds(16 * r, L)] for r in range(2 * ENV_NUM)]
    bvec = b_v[pl.ds(0, L)]

    def blk_body(blk, carry):
        o = blk * L
        rows = o + lax.iota(jnp.int32, L)
        env16 = envs_v[pl.ds(o, L)]
        zero = jnp.zeros((L,), jnp.float32)
        acc_i = zero
        acc_e = zero
        l0 = zero
        l1 = zero
        l2 = zero
        l3 = zero
        envbase = env16 * FACTOR
        for f in range(FACTOR):
            fs = jnp.full((L,), f, jnp.int32)
            fs2 = jnp.full((L,), FACTOR + f, jnp.int32)
            u = plsc.load_gather(urow_v, [rows, fs])
            it = plsc.load_gather(irow_v, [rows, fs])
            p = u * it
            acc_i = acc_i + p
            ue = plsc.load_gather(urow_v, [rows, fs2])
            ie = plsc.load_gather(irow_v, [rows, fs2])
            ee = plsc.load_gather(envt_v, [envbase + f])
            acc_e = acc_e + ue * ie * ee
            h, lane = divmod(f, L)
            l0 = l0 + p * wrows[0 + h][lane]
            l1 = l1 + p * wrows[2 + h][lane]
            l2 = l2 + p * wrows[4 + h][lane]
            l3 = l3 + p * wrows[6 + h][lane]
        inv_v[pl.ds(o, L)] = acc_i
        envsc_v[pl.ds(o, L)] = acc_i + acc_e
        l0 = l0 + bvec[0]
        l1 = l1 + bvec[1]
        l2 = l2 + bvec[2]
        l3 = l3 + bvec[3]
        m = jnp.maximum(jnp.maximum(l0, l1), jnp.maximum(l2, l3))
        t0 = jnp.exp(l0 - m)
        t1 = jnp.exp(l1 - m)
        t2 = jnp.exp(l2 - m)
        t3 = jnp.exp(l3 - m)
        s = t0 + t1 + t2 + t3
        y = _LC0 + s * (_LC1 + _LC2 * s)
        y = y + s * jnp.exp(-y) - 1.0
        y = y + s * jnp.exp(-y) - 1.0
        shift = m + y
        lsmbase = rows * ENV_NUM
        plsc.store_scatter(lsm_v, [lsmbase], l0 - shift)
        plsc.store_scatter(lsm_v, [lsmbase + 1], l1 - shift)
        plsc.store_scatter(lsm_v, [lsmbase + 2], l2 - shift)
        plsc.store_scatter(lsm_v, [lsmbase + 3], l3 - shift)
        return carry

    lax.fori_loop(0, NBLK, blk_body, 0)

    pltpu.sync_copy(inv_v, inv_hbm.at[pl.ds(base, CHUNK)])
    pltpu.sync_copy(envsc_v, envsc_hbm.at[pl.ds(base, CHUNK)])
    pltpu.sync_copy(lsm_v, lsm_hbm.at[pl.ds(base * ENV_NUM, CHUNK * ENV_NUM)])


_sc_call = pl.kernel(
    _sc_body,
    out_type=(
        jax.ShapeDtypeStruct((B,), jnp.float32),
        jax.ShapeDtypeStruct((B,), jnp.float32),
        jax.ShapeDtypeStruct((B * ENV_NUM,), jnp.float32),
    ),
    mesh=plsc.VectorSubcoreMesh(core_axis_name="c", subcore_axis_name="s"),
    scratch_types=[
        pltpu.VMEM((NG, GCH), jnp.int32),
        pltpu.VMEM((NG, GCH), jnp.int32),
        pltpu.VMEM((CHUNK,), jnp.int32),
        pltpu.VMEM((CHUNK, F2), jnp.float32),
        pltpu.VMEM((CHUNK, F2), jnp.float32),
        pltpu.VMEM((ENV_NUM * FACTOR,), jnp.float32),
        pltpu.VMEM((ENV_NUM * FACTOR,), jnp.float32),
        pltpu.VMEM((L,), jnp.float32),
        pltpu.VMEM((CHUNK,), jnp.float32),
        pltpu.VMEM((CHUNK,), jnp.float32),
        pltpu.VMEM((CHUNK * ENV_NUM,), jnp.float32),
        pltpu.SemaphoreType.DMA,
    ],
    compiler_params=pltpu.CompilerParams(
        use_tc_tiling_on_sc=False, needs_layout_passes=False),
)


def kernel(users_id, items_id, envs_id, alpha, emb_user_inv, emb_item_inv,
           emb_user_env, emb_item_env, emb_env, W, b):
    del alpha
    users2 = users_id.reshape(NW * NG, GCH)
    items2 = items_id.reshape(NW * NG, GCH)
    utab = jnp.concatenate([emb_user_inv, emb_user_env], axis=1)
    itab = jnp.concatenate([emb_item_inv, emb_item_env], axis=1)
    w_flat = W.reshape(ENV_NUM * FACTOR)
    b_pad = jnp.pad(b, (0, L - ENV_NUM))
    env_flat = emb_env.reshape(ENV_NUM * FACTOR)
    inv_score, env_score, env_outputs = _sc_call(
        users2, items2, envs_id,
        utab, itab,
        env_flat, w_flat, b_pad)
    return inv_score, env_score, env_outputs.reshape(B, ENV_NUM)


# speedup vs baseline: 1.3056x; 1.3056x over previous
import jax
import jax.numpy as jnp
from jax import lax
from jax.experimental import pallas as pl
from jax.experimental.pallas import tpu as pltpu
from jax.experimental.pallas import tpu_sc as plsc

B = 16384
FACTOR = 32
ENV_NUM = 4
NC = 2
NS = 16
L = 16
NW = NC * NS
CHUNK = B // NW
NBLK = CHUNK // L
GCH = 128
NG = CHUNK // GCH

_LC0 = -0.76336156
_LC1 = 0.9123227
_LC2 = -0.09557938


def _sc_body(users_hbm, items_hbm, envs_hbm,
             uinv_hbm, iinv_hbm, uenv_hbm, ienv_hbm,
             env_hbm, w_hbm, b_hbm,
             inv_hbm, envsc_hbm, lsm_hbm,
             uidx_v, iidx_v, envs_v,
             uinv_v, iinv_v, uenv_v, ienv_v,
             envt_v, w_v, b_v,
             inv_v, envsc_v, lsm_v, sem):
    wid = lax.axis_index("s") * NC + lax.axis_index("c")
    base = wid * CHUNK
    row4 = wid * NG

    for j in range(NG):
        pltpu.sync_copy(users_hbm.at[row4 + j], uidx_v.at[pl.ds(j * GCH, GCH)])
        pltpu.sync_copy(items_hbm.at[row4 + j], iidx_v.at[pl.ds(j * GCH, GCH)])
    pltpu.sync_copy(envs_hbm.at[pl.ds(base, CHUNK)], envs_v)
    pltpu.sync_copy(env_hbm, envt_v)
    pltpu.sync_copy(w_hbm, w_v)
    pltpu.sync_copy(b_hbm, b_v)

    def gather_grp(g, carry):
        o = g * L
        uvec = uidx_v[pl.ds(o, L)] * FACTOR
        ivec = iidx_v[pl.ds(o, L)] * FACTOR
        copies = []
        for j in range(L):
            ur = pl.multiple_of(uvec[j], 8)
            ir = pl.multiple_of(ivec[j], 8)
            dst = pl.ds(pl.multiple_of((o + j) * FACTOR, 8), FACTOR)
            copies.append(pltpu.async_copy(
                uinv_hbm.at[pl.ds(ur, FACTOR)], uinv_v.at[dst], sem))
            copies.append(pltpu.async_copy(
                iinv_hbm.at[pl.ds(ir, FACTOR)], iinv_v.at[dst], sem))
            copies.append(pltpu.async_copy(
                uenv_hbm.at[pl.ds(ur, FACTOR)], uenv_v.at[dst], sem))
            copies.append(pltpu.async_copy(
                ienv_hbm.at[pl.ds(ir, FACTOR)], ienv_v.at[dst], sem))
        for c in copies:
            c.wait()
        return carry

    lax.fori_loop(0, NBLK, gather_grp, 0)

    wrows = [w_v[pl.ds(16 * r, L)] for r in range(2 * ENV_NUM)]
    bvec = b_v[pl.ds(0, L)]

    def blk_body(blk, carry):
        o = blk * L
        rows = o + lax.iota(jnp.int32, L)
        env16 = envs_v[pl.ds(o, L)]
        zero = jnp.zeros((L,), jnp.float32)
        acc_i = zero
        acc_e = zero
        l0 = zero
        l1 = zero
        l2 = zero
        l3 = zero
        rowbase = rows * FACTOR
        envbase = env16 * FACTOR
        for f in range(FACTOR):
            fr = rowbase + f
            u = plsc.load_gather(uinv_v, [fr])
            it = plsc.load_gather(iinv_v, [fr])
            p = u * it
            acc_i = acc_i + p
            ue = plsc.load_gather(uenv_v, [fr])
            ie = plsc.load_gather(ienv_v, [fr])
            ee = plsc.load_gather(envt_v, [envbase + f])
            acc_e = acc_e + ue * ie * ee
            h, lane = divmod(f, L)
            l0 = l0 + p * wrows[0 + h][lane]
            l1 = l1 + p * wrows[2 + h][lane]
            l2 = l2 + p * wrows[4 + h][lane]
            l3 = l3 + p * wrows[6 + h][lane]
        inv_v[pl.ds(o, L)] = acc_i
        envsc_v[pl.ds(o, L)] = acc_i + acc_e
        l0 = l0 + bvec[0]
        l1 = l1 + bvec[1]
        l2 = l2 + bvec[2]
        l3 = l3 + bvec[3]
        m = jnp.maximum(jnp.maximum(l0, l1), jnp.maximum(l2, l3))
        t0 = jnp.exp(l0 - m)
        t1 = jnp.exp(l1 - m)
        t2 = jnp.exp(l2 - m)
        t3 = jnp.exp(l3 - m)
        s = t0 + t1 + t2 + t3
        y = _LC0 + s * (_LC1 + _LC2 * s)
        y = y + s * jnp.exp(-y) - 1.0
        y = y + s * jnp.exp(-y) - 1.0
        shift = m + y
        lsmbase = rows * ENV_NUM
        plsc.store_scatter(lsm_v, [lsmbase], l0 - shift)
        plsc.store_scatter(lsm_v, [lsmbase + 1], l1 - shift)
        plsc.store_scatter(lsm_v, [lsmbase + 2], l2 - shift)
        plsc.store_scatter(lsm_v, [lsmbase + 3], l3 - shift)
        return carry

    lax.fori_loop(0, NBLK, blk_body, 0)

    pltpu.sync_copy(inv_v, inv_hbm.at[pl.ds(base, CHUNK)])
    pltpu.sync_copy(envsc_v, envsc_hbm.at[pl.ds(base, CHUNK)])
    pltpu.sync_copy(lsm_v, lsm_hbm.at[pl.ds(base * ENV_NUM, CHUNK * ENV_NUM)])


_sc_call = pl.kernel(
    _sc_body,
    out_type=(
        jax.ShapeDtypeStruct((B,), jnp.float32),
        jax.ShapeDtypeStruct((B,), jnp.float32),
        jax.ShapeDtypeStruct((B * ENV_NUM,), jnp.float32),
    ),
    mesh=plsc.VectorSubcoreMesh(core_axis_name="c", subcore_axis_name="s"),
    scratch_types=[
        pltpu.VMEM((CHUNK,), jnp.int32),
        pltpu.VMEM((CHUNK,), jnp.int32),
        pltpu.VMEM((CHUNK,), jnp.int32),
        pltpu.VMEM((CHUNK * FACTOR,), jnp.float32),
        pltpu.VMEM((CHUNK * FACTOR,), jnp.float32),
        pltpu.VMEM((CHUNK * FACTOR,), jnp.float32),
        pltpu.VMEM((CHUNK * FACTOR,), jnp.float32),
        pltpu.VMEM((ENV_NUM * FACTOR,), jnp.float32),
        pltpu.VMEM((ENV_NUM * FACTOR,), jnp.float32),
        pltpu.VMEM((L,), jnp.float32),
        pltpu.VMEM((CHUNK,), jnp.float32),
        pltpu.VMEM((CHUNK,), jnp.float32),
        pltpu.VMEM((CHUNK * ENV_NUM,), jnp.float32),
        pltpu.SemaphoreType.DMA,
    ],
    compiler_params=pltpu.CompilerParams(
        use_tc_tiling_on_sc=False, needs_layout_passes=False),
)


def kernel(users_id, items_id, envs_id, alpha, emb_user_inv, emb_item_inv,
           emb_user_env, emb_item_env, emb_env, W, b):
    del alpha
    users2 = users_id.reshape(NW * NG, GCH)
    items2 = items_id.reshape(NW * NG, GCH)
    w_flat = W.reshape(ENV_NUM * FACTOR)
    b_pad = jnp.pad(b, (0, L - ENV_NUM))
    env_flat = emb_env.reshape(ENV_NUM * FACTOR)
    n = emb_user_inv.shape[0] * FACTOR
    inv_score, env_score, env_outputs = _sc_call(
        users2, items2, envs_id,
        emb_user_inv.reshape(n), emb_item_inv.reshape(n),
        emb_user_env.reshape(n), emb_item_env.reshape(n),
        env_flat, w_flat, b_pad)
    return inv_score, env_score, env_outputs.reshape(B, ENV_NUM)


# speedup vs baseline: 2.1531x; 1.6491x over previous
import jax
import jax.numpy as jnp
from jax import lax
from jax.experimental import pallas as pl
from jax.experimental.pallas import tpu as pltpu
from jax.experimental.pallas import tpu_sc as plsc

B = 16384
FACTOR = 32
ENV_NUM = 4
NC = 2
NS = 16
L = 16
NW = NC * NS
CHUNK = B // NW
NBLK = CHUNK // L
GCH = 128
NG = CHUNK // GCH

_LC0 = -0.76336156
_LC1 = 0.9123227
_LC2 = -0.09557938


def _sc_body(users_hbm, items_hbm, envs_hbm,
             uinv_hbm, iinv_hbm, uenv_hbm, ienv_hbm,
             env_hbm, w_hbm, b_hbm,
             inv_hbm, envsc_hbm, lsm_hbm,
             uidx_v, iidx_v, envs_v,
             uinv_v, iinv_v, uenv_v, ienv_v,
             envt_v, w_v, b_v,
             inv_v, envsc_v, lsm_v, sem):
    wid = lax.axis_index("s") * NC + lax.axis_index("c")
    base = wid * CHUNK
    row4 = wid * NG

    for j in range(NG):
        pltpu.sync_copy(users_hbm.at[row4 + j], uidx_v.at[pl.ds(j * GCH, GCH)])
        pltpu.sync_copy(items_hbm.at[row4 + j], iidx_v.at[pl.ds(j * GCH, GCH)])
    pltpu.sync_copy(envs_hbm.at[pl.ds(base, CHUNK)], envs_v)
    pltpu.sync_copy(env_hbm, envt_v)
    pltpu.sync_copy(w_hbm, w_v)
    pltpu.sync_copy(b_hbm, b_v)

    def gather_grp(g, carry):
        o = g * L
        uraw = uidx_v[pl.ds(o, L)]
        iraw = iidx_v[pl.ds(o, L)]
        uvec = (((uraw >> 13) << 18) + ((uraw & 2047) << 7)
                + (((uraw >> 11) & 3) << 5))
        ivec = (((iraw >> 13) << 18) + ((iraw & 2047) << 7)
                + (((iraw >> 11) & 3) << 5))
        copies = []
        for j in range(L):
            ur = pl.multiple_of(uvec[j], 8)
            ir = pl.multiple_of(ivec[j], 8)
            dst = pl.ds(pl.multiple_of((o + j) * FACTOR, 8), FACTOR)
            copies.append(pltpu.async_copy(
                uinv_hbm.at[pl.ds(ur, FACTOR)], uinv_v.at[dst], sem))
            copies.append(pltpu.async_copy(
                iinv_hbm.at[pl.ds(ir, FACTOR)], iinv_v.at[dst], sem))
            copies.append(pltpu.async_copy(
                uenv_hbm.at[pl.ds(ur, FACTOR)], uenv_v.at[dst], sem))
            copies.append(pltpu.async_copy(
                ienv_hbm.at[pl.ds(ir, FACTOR)], ienv_v.at[dst], sem))
        for c in copies:
            c.wait()
        return carry

    lax.fori_loop(0, NBLK, gather_grp, 0)

    wrows = [w_v[pl.ds(16 * r, L)] for r in range(2 * ENV_NUM)]
    bvec = b_v[pl.ds(0, L)]

    def blk_body(blk, carry):
        o = blk * L
        rows = o + lax.iota(jnp.int32, L)
        env16 = envs_v[pl.ds(o, L)]
        zero = jnp.zeros((L,), jnp.float32)
        acc_i = zero
        acc_e = zero
        l0 = zero
        l1 = zero
        l2 = zero
        l3 = zero
        rowbase = rows * FACTOR
        envbase = env16 * FACTOR
        for f in range(FACTOR):
            fr = rowbase + f
            u = plsc.load_gather(uinv_v, [fr])
            it = plsc.load_gather(iinv_v, [fr])
            p = u * it
            acc_i = acc_i + p
            ue = plsc.load_gather(uenv_v, [fr])
            ie = plsc.load_gather(ienv_v, [fr])
            ee = plsc.load_gather(envt_v, [envbase + f])
            acc_e = acc_e + ue * ie * ee
            h, lane = divmod(f, L)
            l0 = l0 + p * wrows[0 + h][lane]
            l1 = l1 + p * wrows[2 + h][lane]
            l2 = l2 + p * wrows[4 + h][lane]
            l3 = l3 + p * wrows[6 + h][lane]
        inv_v[pl.ds(o, L)] = acc_i
        envsc_v[pl.ds(o, L)] = acc_i + acc_e
        l0 = l0 + bvec[0]
        l1 = l1 + bvec[1]
        l2 = l2 + bvec[2]
        l3 = l3 + bvec[3]
        m = jnp.maximum(jnp.maximum(l0, l1), jnp.maximum(l2, l3))
        t0 = jnp.exp(l0 - m)
        t1 = jnp.exp(l1 - m)
        t2 = jnp.exp(l2 - m)
        t3 = jnp.exp(l3 - m)
        s = t0 + t1 + t2 + t3
        y = _LC0 + s * (_LC1 + _LC2 * s)
        y = y + s * jnp.exp(-y) - 1.0
        y = y + s * jnp.exp(-y) - 1.0
        shift = m + y
        lsmbase = rows * ENV_NUM
        plsc.store_scatter(lsm_v, [lsmbase], l0 - shift)
        plsc.store_scatter(lsm_v, [lsmbase + 1], l1 - shift)
        plsc.store_scatter(lsm_v, [lsmbase + 2], l2 - shift)
        plsc.store_scatter(lsm_v, [lsmbase + 3], l3 - shift)
        return carry

    lax.fori_loop(0, NBLK, blk_body, 0)

    pltpu.sync_copy(inv_v, inv_hbm.at[pl.ds(base, CHUNK)])
    pltpu.sync_copy(envsc_v, envsc_hbm.at[pl.ds(base, CHUNK)])
    pltpu.sync_copy(lsm_v, lsm_hbm.at[pl.ds(base * ENV_NUM, CHUNK * ENV_NUM)])


LANE_BLK = 8192
FLAT_BLK = FACTOR * LANE_BLK


def _tc_relayout_body(u_in, i_in, ue_in, ie_in,
                      u_out, i_out, ue_out, ie_out):
    eye = jnp.eye(FACTOR, dtype=jnp.float32)
    quarter = LANE_BLK // 4
    for src, dst in ((u_in, u_out), (i_in, i_out),
                     (ue_in, ue_out), (ie_in, ie_out)):
        xt = src[...].T
        dst[...] = jnp.concatenate(
            [xt[c * quarter:(c + 1) * quarter, :] for c in range(4)], axis=1)


def _make_tc_relayout(n_rows):
    nstep = -(-n_rows // LANE_BLK)
    out2d = jax.ShapeDtypeStruct((nstep * LANE_BLK // 4, 128), jnp.float32)
    return pl.pallas_call(
        _tc_relayout_body,
        grid=(nstep,),
        in_specs=[pl.BlockSpec((FACTOR, LANE_BLK), lambda k: (0, k))] * 4,
        out_specs=[pl.BlockSpec((LANE_BLK // 4, 128),
                                lambda k: (k, 0))] * 4,
        out_shape=[out2d] * 4,
    )


_sc_call = pl.kernel(
    _sc_body,
    out_type=(
        jax.ShapeDtypeStruct((B,), jnp.float32),
        jax.ShapeDtypeStruct((B,), jnp.float32),
        jax.ShapeDtypeStruct((B * ENV_NUM,), jnp.float32),
    ),
    mesh=plsc.VectorSubcoreMesh(core_axis_name="c", subcore_axis_name="s"),
    scratch_types=[
        pltpu.VMEM((CHUNK,), jnp.int32),
        pltpu.VMEM((CHUNK,), jnp.int32),
        pltpu.VMEM((CHUNK,), jnp.int32),
        pltpu.VMEM((CHUNK * FACTOR,), jnp.float32),
        pltpu.VMEM((CHUNK * FACTOR,), jnp.float32),
        pltpu.VMEM((CHUNK * FACTOR,), jnp.float32),
        pltpu.VMEM((CHUNK * FACTOR,), jnp.float32),
        pltpu.VMEM((ENV_NUM * FACTOR,), jnp.float32),
        pltpu.VMEM((ENV_NUM * FACTOR,), jnp.float32),
        pltpu.VMEM((L,), jnp.float32),
        pltpu.VMEM((CHUNK,), jnp.float32),
        pltpu.VMEM((CHUNK,), jnp.float32),
        pltpu.VMEM((CHUNK * ENV_NUM,), jnp.float32),
        pltpu.SemaphoreType.DMA,
    ],
    compiler_params=pltpu.CompilerParams(
        use_tc_tiling_on_sc=False, needs_layout_passes=False),
)


def kernel(users_id, items_id, envs_id, alpha, emb_user_inv, emb_item_inv,
           emb_user_env, emb_item_env, emb_env, W, b):
    del alpha
    users2 = users_id.reshape(NW * NG, GCH)
    items2 = items_id.reshape(NW * NG, GCH)
    w_flat = W.reshape(ENV_NUM * FACTOR)
    b_pad = jnp.pad(b, (0, L - ENV_NUM))
    env_flat = emb_env.reshape(ENV_NUM * FACTOR)
    uinv_f, iinv_f, uenv_f, ienv_f = (
        t.reshape(t.shape[0] * 128)
        for t in _make_tc_relayout(emb_user_inv.shape[0])(
            emb_user_inv.T, emb_item_inv.T, emb_user_env.T, emb_item_env.T))
    inv_score, env_score, env_outputs = _sc_call(
        users2, items2, envs_id,
        uinv_f, iinv_f, uenv_f, ienv_f,
        env_flat, w_flat, b_pad)
    return inv_score, env_score, env_outputs.reshape(B, ENV_NUM)


# speedup vs baseline: 2.1548x; 1.0008x over previous
import jax
import jax.numpy as jnp
from jax import lax
from jax.experimental import pallas as pl
from jax.experimental.pallas import tpu as pltpu
from jax.experimental.pallas import tpu_sc as plsc

B = 16384
FACTOR = 32
ENV_NUM = 4
NC = 2
NS = 16
L = 16
NW = NC * NS
CHUNK = B // NW
NBLK = CHUNK // L
GCH = 128
NG = CHUNK // GCH

_LC0 = -0.76336156
_LC1 = 0.9123227
_LC2 = -0.09557938


def _sc_body(users_hbm, items_hbm, envs_hbm,
             uinv_hbm, iinv_hbm, uenv_hbm, ienv_hbm,
             env_hbm, w_hbm, b_hbm,
             inv_hbm, envsc_hbm, lsm_hbm,
             uidx_v, iidx_v, envs_v,
             uinv_v, iinv_v, uenv_v, ienv_v,
             envt_v, w_v, b_v,
             inv_v, envsc_v, lsm_v, sem):
    wid = lax.axis_index("s") * NC + lax.axis_index("c")
    base = wid * CHUNK
    row4 = wid * NG

    for j in range(NG):
        pltpu.sync_copy(users_hbm.at[row4 + j], uidx_v.at[pl.ds(j * GCH, GCH)])
        pltpu.sync_copy(items_hbm.at[row4 + j], iidx_v.at[pl.ds(j * GCH, GCH)])
    pltpu.sync_copy(envs_hbm.at[pl.ds(base, CHUNK)], envs_v)
    pltpu.sync_copy(env_hbm, envt_v)
    pltpu.sync_copy(w_hbm, w_v)
    pltpu.sync_copy(b_hbm, b_v)

    def gather_grp(g, carry):
        o = g * L
        uraw = uidx_v[pl.ds(o, L)]
        iraw = iidx_v[pl.ds(o, L)]
        uvec = (((uraw >> LB_SHIFT) << (LB_SHIFT + 5))
                + ((uraw & QMASK) << 7)
                + (((uraw >> QK_SHIFT) & 3) << 5))
        ivec = (((iraw >> LB_SHIFT) << (LB_SHIFT + 5))
                + ((iraw & QMASK) << 7)
                + (((iraw >> QK_SHIFT) & 3) << 5))
        copies = []
        for j in range(L):
            ur = pl.multiple_of(uvec[j], 8)
            ir = pl.multiple_of(ivec[j], 8)
            dst = pl.ds(pl.multiple_of((o + j) * FACTOR, 8), FACTOR)
            copies.append(pltpu.async_copy(
                uinv_hbm.at[pl.ds(ur, FACTOR)], uinv_v.at[dst], sem))
            copies.append(pltpu.async_copy(
                iinv_hbm.at[pl.ds(ir, FACTOR)], iinv_v.at[dst], sem))
            copies.append(pltpu.async_copy(
                uenv_hbm.at[pl.ds(ur, FACTOR)], uenv_v.at[dst], sem))
            copies.append(pltpu.async_copy(
                ienv_hbm.at[pl.ds(ir, FACTOR)], ienv_v.at[dst], sem))
        for c in copies:
            c.wait()
        return carry

    lax.fori_loop(0, NBLK, gather_grp, 0)

    wrows = [w_v[pl.ds(16 * r, L)] for r in range(2 * ENV_NUM)]
    bvec = b_v[pl.ds(0, L)]

    def blk_body(blk, carry):
        o = blk * L
        rows = o + lax.iota(jnp.int32, L)
        env16 = envs_v[pl.ds(o, L)]
        zero = jnp.zeros((L,), jnp.float32)
        acc_i = zero
        acc_e = zero
        l0 = zero
        l1 = zero
        l2 = zero
        l3 = zero
        rowbase = rows * FACTOR
        envbase = env16 * FACTOR
        for f in range(FACTOR):
            fr = rowbase + f
            u = plsc.load_gather(uinv_v, [fr])
            it = plsc.load_gather(iinv_v, [fr])
            p = u * it
            acc_i = acc_i + p
            ue = plsc.load_gather(uenv_v, [fr])
            ie = plsc.load_gather(ienv_v, [fr])
            ee = plsc.load_gather(envt_v, [envbase + f])
            acc_e = acc_e + ue * ie * ee
            h, lane = divmod(f, L)
            l0 = l0 + p * wrows[0 + h][lane]
            l1 = l1 + p * wrows[2 + h][lane]
            l2 = l2 + p * wrows[4 + h][lane]
            l3 = l3 + p * wrows[6 + h][lane]
        inv_v[pl.ds(o, L)] = acc_i
        envsc_v[pl.ds(o, L)] = acc_i + acc_e
        l0 = l0 + bvec[0]
        l1 = l1 + bvec[1]
        l2 = l2 + bvec[2]
        l3 = l3 + bvec[3]
        m = jnp.maximum(jnp.maximum(l0, l1), jnp.maximum(l2, l3))
        t0 = jnp.exp(l0 - m)
        t1 = jnp.exp(l1 - m)
        t2 = jnp.exp(l2 - m)
        t3 = jnp.exp(l3 - m)
        s = t0 + t1 + t2 + t3
        y = _LC0 + s * (_LC1 + _LC2 * s)
        y = y + s * jnp.exp(-y) - 1.0
        y = y + s * jnp.exp(-y) - 1.0
        shift = m + y
        lsmbase = rows * ENV_NUM
        plsc.store_scatter(lsm_v, [lsmbase], l0 - shift)
        plsc.store_scatter(lsm_v, [lsmbase + 1], l1 - shift)
        plsc.store_scatter(lsm_v, [lsmbase + 2], l2 - shift)
        plsc.store_scatter(lsm_v, [lsmbase + 3], l3 - shift)
        return carry

    lax.fori_loop(0, NBLK, blk_body, 0)

    pltpu.sync_copy(inv_v, inv_hbm.at[pl.ds(base, CHUNK)])
    pltpu.sync_copy(envsc_v, envsc_hbm.at[pl.ds(base, CHUNK)])
    pltpu.sync_copy(lsm_v, lsm_hbm.at[pl.ds(base * ENV_NUM, CHUNK * ENV_NUM)])


LANE_BLK = 8192
FLAT_BLK = FACTOR * LANE_BLK
LB_SHIFT = LANE_BLK.bit_length() - 1
QK_SHIFT = LB_SHIFT - 2
QMASK = (LANE_BLK // 4) - 1


def _tc_relayout_body(u_in, i_in, ue_in, ie_in,
                      u_out, i_out, ue_out, ie_out):
    eye = jnp.eye(FACTOR, dtype=jnp.float32)
    quarter = LANE_BLK // 4
    for src, dst in ((u_in, u_out), (i_in, i_out),
                     (ue_in, ue_out), (ie_in, ie_out)):
        xt = src[...].T
        dst[...] = jnp.concatenate(
            [xt[c * quarter:(c + 1) * quarter, :] for c in range(4)], axis=1)


def _make_tc_relayout(n_rows):
    nstep = -(-n_rows // LANE_BLK)
    out2d = jax.ShapeDtypeStruct((nstep * LANE_BLK // 4, 128), jnp.float32)
    return pl.pallas_call(
        _tc_relayout_body,
        grid=(nstep,),
        in_specs=[pl.BlockSpec((FACTOR, LANE_BLK), lambda k: (0, k))] * 4,
        out_specs=[pl.BlockSpec((LANE_BLK // 4, 128),
                                lambda k: (k, 0))] * 4,
        out_shape=[out2d] * 4,
    )


_sc_call = pl.kernel(
    _sc_body,
    out_type=(
        jax.ShapeDtypeStruct((B,), jnp.float32),
        jax.ShapeDtypeStruct((B,), jnp.float32),
        jax.ShapeDtypeStruct((B * ENV_NUM,), jnp.float32),
    ),
    mesh=plsc.VectorSubcoreMesh(core_axis_name="c", subcore_axis_name="s"),
    scratch_types=[
        pltpu.VMEM((CHUNK,), jnp.int32),
        pltpu.VMEM((CHUNK,), jnp.int32),
        pltpu.VMEM((CHUNK,), jnp.int32),
        pltpu.VMEM((CHUNK * FACTOR,), jnp.float32),
        pltpu.VMEM((CHUNK * FACTOR,), jnp.float32),
        pltpu.VMEM((CHUNK * FACTOR,), jnp.float32),
        pltpu.VMEM((CHUNK * FACTOR,), jnp.float32),
        pltpu.VMEM((ENV_NUM * FACTOR,), jnp.float32),
        pltpu.VMEM((ENV_NUM * FACTOR,), jnp.float32),
        pltpu.VMEM((L,), jnp.float32),
        pltpu.VMEM((CHUNK,), jnp.float32),
        pltpu.VMEM((CHUNK,), jnp.float32),
        pltpu.VMEM((CHUNK * ENV_NUM,), jnp.float32),
        pltpu.SemaphoreType.DMA,
    ],
    compiler_params=pltpu.CompilerParams(
        use_tc_tiling_on_sc=False, needs_layout_passes=False),
)


def kernel(users_id, items_id, envs_id, alpha, emb_user_inv, emb_item_inv,
           emb_user_env, emb_item_env, emb_env, W, b):
    del alpha
    users2 = users_id.reshape(NW * NG, GCH)
    items2 = items_id.reshape(NW * NG, GCH)
    w_flat = W.reshape(ENV_NUM * FACTOR)
    b_pad = jnp.pad(b, (0, L - ENV_NUM))
    env_flat = emb_env.reshape(ENV_NUM * FACTOR)
    uinv_f, iinv_f, uenv_f, ienv_f = (
        t.reshape(t.shape[0] * 128)
        for t in _make_tc_relayout(emb_user_inv.shape[0])(
            emb_user_inv.T, emb_item_inv.T, emb_user_env.T, emb_item_env.T))
    inv_score, env_score, env_outputs = _sc_call(
        users2, items2, envs_id,
        uinv_f, iinv_f, uenv_f, ienv_f,
        env_flat, w_flat, b_pad)
    return inv_score, env_score, env_outputs.reshape(B, ENV_NUM)
